# baseline (speedup 1.0000x reference)
import functools

import jax
import jax.numpy as jnp
import numpy as np
from jax import lax
from jax.experimental import pallas as pl
from jax.experimental.pallas import tpu as pltpu
from jax.experimental.pallas import tpu_sc as plsc

V = 100000
B = 1024
_LS = 0.1
_S = np.float32(_LS / (V - 2))
_C = np.float32(1.0 - _LS)
_KL_CONST = np.float32((V - 1) * (_S * np.float32(np.log(_S))) + _C * np.float32(np.log(_C)))
_CMS = np.float32(_C - _S)

_VT = 60000
_VS = V - _VT

_NC = 2
_NS = 16
_L = 16
_NW = _NC * _NS
_NVS = 4
_NCB = _NW // _NVS
_SPT = _VS // _NVS
_SCR = 200
_SNCH = _SPT // _SCR

_sc_mesh = plsc.VectorSubcoreMesh(core_axis_name="c", subcore_axis_name="s")


@functools.partial(
    pl.kernel,
    mesh=_sc_mesh,
    out_type=[
        jax.ShapeDtypeStruct((_NVS, B), jnp.float32),
        jax.ShapeDtypeStruct((_NVS, B), jnp.float32),
    ],
    scratch_types=[
        pltpu.VMEM((_SCR, 128), jnp.float32),
        pltpu.VMEM((_SCR, 128), jnp.float32),
        pltpu.VMEM((128,), jnp.int32),
        pltpu.VMEM((128,), jnp.float32),
        pltpu.VMEM((128,), jnp.float32),
        pltpu.SemaphoreType.DMA,
        pltpu.SemaphoreType.DMA,
    ],
)
def _sc_nmt(xt_hbm, tgt_hbm, rs_hbm, val_hbm, buf0, buf1, tgt_v, orow_v,
            oval_v, sem0, sem1):
    wid = lax.axis_index("s") * _NC + lax.axis_index("c")
    vs = lax.rem(wid, _NVS)
    cb = wid // _NVS
    v0 = _VT + vs * _SPT
    c0 = cb * 128

    bufs = (buf0, buf1)
    sems = (sem0, sem1)

    def chunk_copy(k, b):
        return pltpu.make_async_copy(
            xt_hbm.at[pl.ds(v0 + k * _SCR, _SCR), pl.ds(c0, 128)],
            bufs[b],
            sems[b],
        )

    pltpu.sync_copy(tgt_hbm.at[pl.ds(c0, 128)], tgt_v)
    tgts = [tgt_v[pl.ds(g * _L, _L)] for g in range(8)]

    chunk_copy(0, 0).start()
    chunk_copy(1, 1).start()

    zero = jnp.zeros((_L,), jnp.float32)

    def pair_body(p, carry):
        accs = carry
        for b in range(2):
            k = p * 2 + b
            chunk_copy(k, b).wait()
            buf = bufs[b]
            vbase = v0 + k * _SCR

            def row_body(r, accs):
                accs = list(accs)
                vrow = vbase + r
                for g in range(8):
                    x = buf[r, pl.ds(g * _L, _L)]
                    accs[g] = accs[g] + x
                    accs[8 + g] = accs[8 + g] + jnp.where(
                        tgts[g] == vrow, x, jnp.float32(0)
                    )
                return tuple(accs)

            accs = lax.fori_loop(0, _SCR, row_body, accs)

            @pl.when(k + 2 < _SNCH)
            def _():
                chunk_copy(k + 2, b).start()

        return accs

    accs = lax.fori_loop(0, _SNCH // 2, pair_body, tuple([zero] * 16))

    for g in range(8):
        orow_v[pl.ds(g * _L, _L)] = accs[g]
        oval_v[pl.ds(g * _L, _L)] = accs[8 + g]
    pltpu.sync_copy(orow_v, rs_hbm.at[vs, pl.ds(c0, 128)])
    pltpu.sync_copy(oval_v, val_hbm.at[vs, pl.ds(c0, 128)])


_RC = 1000
_K = 4
_TNCH = _VT // _RC
_NROUND = _TNCH // _K


def _tc_body(xt_hbm, t_ref, rs_ref, val_ref, buf, sem):
    def copy(c, b):
        return pltpu.make_async_copy(
            xt_hbm.at[pl.ds(c * _RC, _RC)], buf.at[b], sem.at[b]
        )

    for b in range(_K):
        copy(b, b).start()

    def round_body(r, carry):
        acc, vacc = carry
        for b in range(_K):
            c = r * _K + b
            copy(c, b).wait()
            x = buf[b]
            acc = acc + jnp.sum(x, axis=0, keepdims=True)
            rows = lax.broadcasted_iota(jnp.int32, (_RC, B), 0)
            tsh = t_ref[...] - c * _RC
            vacc = vacc + jnp.sum(
                jnp.where(rows == tsh, x, jnp.float32(0)), axis=0, keepdims=True
            )

            @pl.when(r + 1 < _NROUND)
            def _():
                copy(c + _K, b).start()

        return acc, vacc

    acc, vacc = lax.fori_loop(
        0,
        _NROUND,
        round_body,
        (jnp.zeros((1, B), jnp.float32), jnp.zeros((1, B), jnp.float32)),
    )
    rs_ref[...] = acc
    val_ref[...] = vacc


_tc_call = pl.pallas_call(
    _tc_body,
    in_specs=[
        pl.BlockSpec(memory_space=pl.ANY),
        pl.BlockSpec(memory_space=pltpu.VMEM),
    ],
    out_specs=[
        pl.BlockSpec(memory_space=pltpu.VMEM),
        pl.BlockSpec(memory_space=pltpu.VMEM),
    ],
    out_shape=[
        jax.ShapeDtypeStruct((1, B), jnp.float32),
        jax.ShapeDtypeStruct((1, B), jnp.float32),
    ],
    scratch_shapes=[
        pltpu.VMEM((_K, _RC, B), jnp.float32),
        pltpu.SemaphoreType.DMA((_K,)),
    ],
)


def _combine_body(rt_ref, vt_ref, rsc_ref, vsc_ref, o_ref):
    rs = rt_ref[...] + jnp.sum(rsc_ref[...], axis=0, keepdims=True)
    val = vt_ref[...] + jnp.sum(vsc_ref[...], axis=0, keepdims=True)
    o_ref[...] = _KL_CONST - _S * rs - _CMS * val


_combine = pl.pallas_call(
    _combine_body,
    out_shape=jax.ShapeDtypeStruct((1, B), jnp.float32),
)


def kernel(output, target):
    tgt = target.astype(jnp.int32)
    xt = output.T
    rs_sc, val_sc = _sc_nmt(xt, tgt)
    rs_tc, val_tc = _tc_call(xt, tgt.reshape(1, B))
    res = _combine(rs_tc, val_tc, rs_sc, val_sc)
    return res.reshape(B)


# speedup vs baseline: 1.0111x; 1.0111x over previous
import functools

import jax
import jax.numpy as jnp
import numpy as np
from jax import lax
from jax.experimental import pallas as pl
from jax.experimental.pallas import tpu as pltpu
from jax.experimental.pallas import tpu_sc as plsc

V = 100000
B = 1024
_LS = 0.1
_S = np.float32(_LS / (V - 2))
_C = np.float32(1.0 - _LS)
_KL_CONST = np.float32((V - 1) * (_S * np.float32(np.log(_S))) + _C * np.float32(np.log(_C)))
_CMS = np.float32(_C - _S)

_VT = 68000
_VS = V - _VT

_NC = 2
_NS = 16
_L = 16
_NW = _NC * _NS
_NVS = 4
_NCB = _NW // _NVS
_SPT = _VS // _NVS
_SCR = 200
_SNCH = _SPT // _SCR

_sc_mesh = plsc.VectorSubcoreMesh(core_axis_name="c", subcore_axis_name="s")


@functools.partial(
    pl.kernel,
    mesh=_sc_mesh,
    out_type=[
        jax.ShapeDtypeStruct((_NVS, B), jnp.float32),
        jax.ShapeDtypeStruct((_NVS, B), jnp.float32),
    ],
    scratch_types=[
        pltpu.VMEM((_SCR, 128), jnp.float32),
        pltpu.VMEM((_SCR, 128), jnp.float32),
        pltpu.VMEM((128,), jnp.int32),
        pltpu.VMEM((128,), jnp.float32),
        pltpu.VMEM((128,), jnp.float32),
        pltpu.SemaphoreType.DMA,
        pltpu.SemaphoreType.DMA,
    ],
)
def _sc_nmt(xt_hbm, tgt_hbm, rs_hbm, val_hbm, buf0, buf1, tgt_v, orow_v,
            oval_v, sem0, sem1):
    wid = lax.axis_index("s") * _NC + lax.axis_index("c")
    vs = lax.rem(wid, _NVS)
    cb = wid // _NVS
    v0 = _VT + vs * _SPT
    c0 = cb * 128

    bufs = (buf0, buf1)
    sems = (sem0, sem1)

    def chunk_copy(k, b):
        return pltpu.make_async_copy(
            xt_hbm.at[pl.ds(v0 + k * _SCR, _SCR), pl.ds(c0, 128)],
            bufs[b],
            sems[b],
        )

    pltpu.sync_copy(tgt_hbm.at[pl.ds(c0, 128)], tgt_v)
    tgts = [tgt_v[pl.ds(g * _L, _L)] for g in range(8)]

    chunk_copy(0, 0).start()
    chunk_copy(1, 1).start()

    zero = jnp.zeros((_L,), jnp.float32)

    def pair_body(p, carry):
        accs = carry
        for b in range(2):
            k = p * 2 + b
            chunk_copy(k, b).wait()
            buf = bufs[b]
            vbase = v0 + k * _SCR

            def row_body(r, accs):
                accs = list(accs)
                vrow = vbase + r
                for g in range(8):
                    x = buf[r, pl.ds(g * _L, _L)]
                    accs[g] = accs[g] + x
                    accs[8 + g] = accs[8 + g] + jnp.where(
                        tgts[g] == vrow, x, jnp.float32(0)
                    )
                return tuple(accs)

            accs = lax.fori_loop(0, _SCR, row_body, accs)

            @pl.when(k + 2 < _SNCH)
            def _():
                chunk_copy(k + 2, b).start()

        return accs

    accs = lax.fori_loop(0, _SNCH // 2, pair_body, tuple([zero] * 16))

    for g in range(8):
        orow_v[pl.ds(g * _L, _L)] = accs[g]
        oval_v[pl.ds(g * _L, _L)] = accs[8 + g]
    pltpu.sync_copy(orow_v, rs_hbm.at[vs, pl.ds(c0, 128)])
    pltpu.sync_copy(oval_v, val_hbm.at[vs, pl.ds(c0, 128)])


_RC = 1000
_K = 4
_TNCH = _VT // _RC
_NROUND = _TNCH // _K


def _tc_body(xt_hbm, t_ref, rs_ref, val_ref, buf, sem):
    def copy(c, b):
        return pltpu.make_async_copy(
            xt_hbm.at[pl.ds(c * _RC, _RC)], buf.at[b], sem.at[b]
        )

    for b in range(_K):
        copy(b, b).start()

    def round_body(r, carry):
        acc, vacc = carry
        for b in range(_K):
            c = r * _K + b
            copy(c, b).wait()
            x = buf[b]
            acc = acc + jnp.sum(x, axis=0, keepdims=True)
            rows = lax.broadcasted_iota(jnp.int32, (_RC, B), 0)
            tsh = t_ref[...] - c * _RC
            vacc = vacc + jnp.sum(
                jnp.where(rows == tsh, x, jnp.float32(0)), axis=0, keepdims=True
            )

            @pl.when(r + 1 < _NROUND)
            def _():
                copy(c + _K, b).start()

        return acc, vacc

    acc, vacc = lax.fori_loop(
        0,
        _NROUND,
        round_body,
        (jnp.zeros((1, B), jnp.float32), jnp.zeros((1, B), jnp.float32)),
    )
    rs_ref[...] = acc
    val_ref[...] = vacc


_tc_call = pl.pallas_call(
    _tc_body,
    in_specs=[
        pl.BlockSpec(memory_space=pl.ANY),
        pl.BlockSpec(memory_space=pltpu.VMEM),
    ],
    out_specs=[
        pl.BlockSpec(memory_space=pltpu.VMEM),
        pl.BlockSpec(memory_space=pltpu.VMEM),
    ],
    out_shape=[
        jax.ShapeDtypeStruct((1, B), jnp.float32),
        jax.ShapeDtypeStruct((1, B), jnp.float32),
    ],
    scratch_shapes=[
        pltpu.VMEM((_K, _RC, B), jnp.float32),
        pltpu.SemaphoreType.DMA((_K,)),
    ],
)


def _combine_body(rt_ref, vt_ref, rsc_ref, vsc_ref, o_ref):
    rs = rt_ref[...] + jnp.sum(rsc_ref[...], axis=0, keepdims=True)
    val = vt_ref[...] + jnp.sum(vsc_ref[...], axis=0, keepdims=True)
    o_ref[...] = _KL_CONST - _S * rs - _CMS * val


_combine = pl.pallas_call(
    _combine_body,
    out_shape=jax.ShapeDtypeStruct((1, B), jnp.float32),
)


def kernel(output, target):
    tgt = target.astype(jnp.int32)
    xt = output.T
    rs_sc, val_sc = _sc_nmt(xt, tgt)
    rs_tc, val_tc = _tc_call(xt, tgt.reshape(1, B))
    res = _combine(rs_tc, val_tc, rs_sc, val_sc)
    return res.reshape(B)
